# baseline (speedup 1.0000x reference)
import functools
import math

import jax
import jax.numpy as jnp
from jax import lax
from jax.experimental import pallas as pl
from jax.experimental.pallas import tpu as pltpu
from jax.experimental.pallas import tpu_sc as plsc

VOCAB = 1000000
D = 64
ROWS = 4096
COLS = 200
B = ROWS * COLS
NC = 2
NS = 16
NW = NC * NS
BPW = B // NW
G = 128
NG = BPW // G
L = 16
SCALE = math.sqrt(D)

_mesh = plsc.VectorSubcoreMesh(
    core_axis_name="c", subcore_axis_name="s", num_cores=NC, num_subcores=NS
)


def _body(x_hbm, table_hbm, out_hbm, idx_v, rows_v, gsem):
    wid = lax.axis_index("s") * NC + lax.axis_index("c")
    pltpu.sync_copy(x_hbm.at[wid], idx_v)

    def step(g, _):
        pltpu.async_copy(table_hbm.at[idx_v.at[g]], rows_v, gsem).wait()

        @plsc.parallel_loop(0, G, unroll=4)
        def _add(r):
            for c in range(D // L):
                sl = pl.ds(c * L, L)
                rows_v[r, sl] = rows_v[r, sl] + SCALE

        pltpu.sync_copy(rows_v, out_hbm.at[wid, g])
        return 0

    lax.fori_loop(0, NG, step, 0)


@jax.jit
def _embed(xr, table):
    k = pl.kernel(
        _body,
        out_type=jax.ShapeDtypeStruct((NW, NG, G, D), jnp.float32),
        mesh=_mesh,
        compiler_params=pltpu.CompilerParams(use_tc_tiling_on_sc=False),
        scratch_types=[
            pltpu.VMEM((NG, G), jnp.int32),
            pltpu.VMEM((G, D), jnp.float32),
            pltpu.SemaphoreType.DMA,
        ],
    )
    return k(xr, table)


def kernel(x, table):
    xr = x.reshape(NW, NG, G)
    out = _embed(xr, table)
    return out.reshape(ROWS, COLS, D)


# speedup vs baseline: 1.1463x; 1.1463x over previous
import math

import jax
import jax.numpy as jnp
from jax import lax
from jax.experimental import pallas as pl
from jax.experimental.pallas import tpu as pltpu
from jax.experimental.pallas import tpu_sc as plsc

VOCAB = 1000000
D = 64
ROWS = 4096
COLS = 200
B = ROWS * COLS
NC = 2
NS = 16
NW = NC * NS
BPW = B // NW
G = 128
NG = BPW // G
L = 16
SCALE = math.sqrt(D)

_mesh = plsc.VectorSubcoreMesh(
    core_axis_name="c", subcore_axis_name="s", num_cores=NC, num_subcores=NS
)


def _body(x_hbm, table_hbm, out_hbm, idx_v, in_v, out_v, gsem, ssem):
    wid = lax.axis_index("s") * NC + lax.axis_index("c")
    pltpu.sync_copy(x_hbm.at[wid], idx_v)

    def start_gather(g, b):
        pltpu.async_copy(table_hbm.at[idx_v.at[g]], in_v.at[b], gsem.at[b])

    def wait_gather(g, b):
        pltpu.make_async_copy(table_hbm.at[idx_v.at[g]], in_v.at[b], gsem.at[b]).wait()

    def start_store(g, b):
        pltpu.async_copy(out_v.at[b], out_hbm.at[wid, g], ssem.at[b])

    def wait_store(g, b):
        pltpu.make_async_copy(out_v.at[b], out_hbm.at[wid, g], ssem.at[b]).wait()

    def add_group(b):
        @plsc.parallel_loop(0, G, unroll=4)
        def _add(r):
            for c in range(D // L):
                sl = pl.ds(c * L, L)
                out_v[b, r, sl] = in_v[b, r, sl] + SCALE

    for b in range(2):
        start_gather(b, b)

    for b in range(2):
        wait_gather(b, b)
        add_group(b)
        start_gather(b + 2, b)
        start_store(b, b)

    def step2(i, _):
        g0 = 2 + i * 2
        for b in range(2):
            g = g0 + b
            wait_gather(g, b)
            wait_store(g - 2, b)
            add_group(b)

            @pl.when(g + 2 < NG)
            def _():
                start_gather(g + 2, b)

            start_store(g, b)
        return 0

    lax.fori_loop(0, (NG - 2) // 2, step2, 0)

    for b in range(2):
        wait_store(NG - 2 + b, b)


@jax.jit
def _embed(xr, table):
    k = pl.kernel(
        _body,
        out_type=jax.ShapeDtypeStruct((NW, NG, G, D), jnp.float32),
        mesh=_mesh,
        compiler_params=pltpu.CompilerParams(use_tc_tiling_on_sc=False),
        scratch_types=[
            pltpu.VMEM((NG, G), jnp.int32),
            pltpu.VMEM((2, G, D), jnp.float32),
            pltpu.VMEM((2, G, D), jnp.float32),
            pltpu.SemaphoreType.DMA((2,)),
            pltpu.SemaphoreType.DMA((2,)),
        ],
    )
    return k(xr, table)


def kernel(x, table):
    xr = x.reshape(NW, NG, G)
    out = _embed(xr, table)
    return out.reshape(ROWS, COLS, D)


# speedup vs baseline: 1.1617x; 1.0134x over previous
import math

import jax
import jax.numpy as jnp
from jax import lax
from jax.experimental import pallas as pl
from jax.experimental.pallas import tpu as pltpu
from jax.experimental.pallas import tpu_sc as plsc

VOCAB = 1000000
D = 64
ROWS = 4096
COLS = 200
NC = 2
NS = 16
NW = NC * NS
RPW = ROWS // NW
G1 = 128
G2 = COLS - G1
L = 16
SCALE = math.sqrt(D)

_mesh = plsc.VectorSubcoreMesh(
    core_axis_name="c", subcore_axis_name="s", num_cores=NC, num_subcores=NS
)


def _body(x_hbm, table_hbm, out_hbm, idx_v, in_v, out_v, gsem, ssem):
    wid = lax.axis_index("s") * NC + lax.axis_index("c")
    row0 = wid * RPW
    pltpu.sync_copy(x_hbm.at[pl.ds(row0, RPW)], idx_v)

    def start_gather(r, b):
        pltpu.async_copy(
            table_hbm.at[idx_v.at[r, pl.ds(0, G1)]],
            in_v.at[b, pl.ds(0, G1)], gsem.at[b])
        pltpu.async_copy(
            table_hbm.at[idx_v.at[r, pl.ds(G1, G2)]],
            in_v.at[b, pl.ds(G1, G2)], gsem.at[b])

    def wait_gather(r, b):
        pltpu.make_async_copy(
            table_hbm.at[idx_v.at[r, pl.ds(0, G1)]],
            in_v.at[b, pl.ds(0, G1)], gsem.at[b]).wait()
        pltpu.make_async_copy(
            table_hbm.at[idx_v.at[r, pl.ds(G1, G2)]],
            in_v.at[b, pl.ds(G1, G2)], gsem.at[b]).wait()

    def start_store(r, b):
        pltpu.async_copy(out_v.at[b], out_hbm.at[row0 + r], ssem.at[b])

    def wait_store(r, b):
        pltpu.make_async_copy(out_v.at[b], out_hbm.at[row0 + r], ssem.at[b]).wait()

    def add_row(b):
        @plsc.parallel_loop(0, COLS, unroll=4)
        def _add(c):
            for v in range(D // L):
                sl = pl.ds(v * L, L)
                out_v[b, c, sl] = in_v[b, c, sl] + SCALE

    for b in range(2):
        start_gather(b, b)

    for b in range(2):
        wait_gather(b, b)
        add_row(b)
        start_gather(b + 2, b)
        start_store(b, b)

    def step2(i, _):
        r0 = 2 + i * 2
        for b in range(2):
            r = r0 + b
            wait_gather(r, b)
            wait_store(r - 2, b)
            add_row(b)

            @pl.when(r + 2 < RPW)
            def _():
                start_gather(r + 2, b)

            start_store(r, b)
        return 0

    lax.fori_loop(0, (RPW - 2) // 2, step2, 0)

    for b in range(2):
        wait_store(RPW - 2 + b, b)


@jax.jit
def _embed(x, table):
    k = pl.kernel(
        _body,
        out_type=jax.ShapeDtypeStruct((ROWS, COLS, D), jnp.float32),
        mesh=_mesh,
        compiler_params=pltpu.CompilerParams(use_tc_tiling_on_sc=False),
        scratch_types=[
            pltpu.VMEM((RPW, COLS), jnp.int32),
            pltpu.VMEM((2, COLS, D), jnp.float32),
            pltpu.VMEM((2, COLS, D), jnp.float32),
            pltpu.SemaphoreType.DMA((2,)),
            pltpu.SemaphoreType.DMA((2,)),
        ],
    )
    return k(x, table)


def kernel(x, table):
    return _embed(x, table)


# speedup vs baseline: 1.2003x; 1.0332x over previous
import math

import jax
import jax.numpy as jnp
from jax import lax
from jax.experimental import pallas as pl
from jax.experimental.pallas import tpu as pltpu
from jax.experimental.pallas import tpu_sc as plsc

VOCAB = 1000000
D = 64
ROWS = 4096
COLS = 200
NC = 2
NS = 16
NW = NC * NS
TPW = ROWS // NW
L = 16
SCALE = math.sqrt(D)

_mesh = plsc.VectorSubcoreMesh(
    core_axis_name="c", subcore_axis_name="s", num_cores=NC, num_subcores=NS
)


def _body(x_hbm, tab_hbm, out_hbm, xq_v, in_v, out_v, gsem, ssem):
    wid = lax.axis_index("s") * NC + lax.axis_index("c")
    a0 = wid * TPW
    pltpu.sync_copy(x_hbm.at[:, pl.ds(a0, TPW)], xq_v)

    def start_gather(s, b):
        pltpu.async_copy(tab_hbm.at[xq_v.at[s]], in_v.at[b], gsem.at[b])

    def wait_gather(s, b):
        pltpu.make_async_copy(tab_hbm.at[xq_v.at[s]], in_v.at[b], gsem.at[b]).wait()

    def start_store(s, b):
        pltpu.async_copy(out_v.at[b], out_hbm.at[s, pl.ds(0, D // 8), wid], ssem.at[b])

    def wait_store(s, b):
        pltpu.make_async_copy(
            out_v.at[b], out_hbm.at[s, pl.ds(0, D // 8), wid], ssem.at[b]).wait()

    def transpose_add(s, b):
        @plsc.parallel_loop(0, (TPW // L) * (D // 8), unroll=2)
        def _chunk(i):
            p = i >> 3
            c1 = i & 7
            rows = lax.broadcasted_iota(jnp.int32, (L,), 0) + p * L
            for c0 in range(8):
                cols = jnp.full((L,), c0, jnp.int32) + c1 * 8
                vals = plsc.load_gather(in_v.at[b], [rows, cols])
                out_v[b, c1, c0, pl.ds(p * L, L)] = vals + SCALE


    for b in range(2):
        start_gather(b, b)

    def main(i, _):
        s0 = i * 4
        for j in range(4):
            s = s0 + j

            @pl.when(s >= 2)
            def _():
                start_store(s - 2, (j + 2) % 4)

            @pl.when(s + 2 < COLS)
            def _():
                start_gather(s + 2, (j + 2) % 4)

            wait_gather(s, j)

            @pl.when(s >= 4)
            def _():
                wait_store(s - 4, j)

            transpose_add(s, j)
        return 0

    lax.fori_loop(0, COLS // 4, main, 0)

    for s in range(COLS - 2, COLS):
        start_store(s, s % 4)
    for s in range(COLS - 4, COLS):
        wait_store(s, s % 4)


@jax.jit
def _embed(xt, tab):
    k = pl.kernel(
        _body,
        out_type=jax.ShapeDtypeStruct((COLS, D // 8, ROWS // TPW, 8, TPW), jnp.float32),
        mesh=_mesh,
        compiler_params=pltpu.CompilerParams(
            use_tc_tiling_on_sc=False, needs_layout_passes=False),
        scratch_types=[
            pltpu.VMEM((COLS, TPW), jnp.int32),
            pltpu.VMEM((4, TPW, 2 * D), jnp.float32),
            pltpu.VMEM((4, D // 8, 8, TPW), jnp.float32),
            pltpu.SemaphoreType.DMA((4,)),
            pltpu.SemaphoreType.DMA((4,)),
        ],
    )
    return k(xt, tab)


def kernel(x, table):
    xt = x.T
    tab = jnp.pad(table, ((0, 0), (0, D)))
    out6 = _embed(xt, tab)
    return out6.transpose(2, 4, 0, 1, 3).reshape(ROWS, COLS, D)


# speedup vs baseline: 1.2065x; 1.0052x over previous
import math

import jax
import jax.numpy as jnp
from jax import lax
from jax.experimental import pallas as pl
from jax.experimental.pallas import tpu as pltpu
from jax.experimental.pallas import tpu_sc as plsc

VOCAB = 1000000
D = 64
ROWS = 4096
COLS = 200
NC = 2
NS = 16
NW = NC * NS
TPW = ROWS // NW
L = 16
SCALE = math.sqrt(D)

_mesh = plsc.VectorSubcoreMesh(
    core_axis_name="c", subcore_axis_name="s", num_cores=NC, num_subcores=NS
)


def _body(x_hbm, tab_hbm, out_hbm, xq_v, in_v, out_v, gsem, ssem):
    wid = lax.axis_index("s") * NC + lax.axis_index("c")
    a0 = wid * TPW
    pltpu.sync_copy(x_hbm.at[:, pl.ds(a0, TPW)], xq_v)

    def start_gather(s, b):
        pltpu.async_copy(
            tab_hbm.at[xq_v.at[s]], in_v.at[b], gsem.at[b])

    def wait_gather(s, b):
        pltpu.make_async_copy(
            tab_hbm.at[xq_v.at[s]], in_v.at[b], gsem.at[b]).wait()

    def start_store(s, b):
        pltpu.async_copy(out_v.at[b], out_hbm.at[s, pl.ds(0, D // 8), wid], ssem.at[b])

    def wait_store(s, b):
        pltpu.make_async_copy(
            out_v.at[b], out_hbm.at[s, pl.ds(0, D // 8), wid], ssem.at[b]).wait()

    def transpose_add(s, b):
        @plsc.parallel_loop(0, (TPW // L) * (D // 8), unroll=4)
        def _chunk(i):
            p = i >> 3
            c1 = i & 7
            rows = lax.broadcasted_iota(jnp.int32, (L,), 0) + p * L
            for c0 in range(8):
                cols = jnp.full((L,), c0, jnp.int32) + c1 * 8
                vals = plsc.load_gather(in_v.at[b], [rows, cols])
                out_v[b, c1, c0, pl.ds(p * L, L)] = vals + SCALE


    for b in range(2):
        start_gather(b, b)

    def main(i, _):
        s0 = i * 4
        for j in range(4):
            s = s0 + j

            @pl.when(s >= 2)
            def _():
                start_store(s - 2, (j + 2) % 4)

            @pl.when(s + 2 < COLS)
            def _():
                start_gather(s + 2, (j + 2) % 4)

            wait_gather(s, j)

            @pl.when(s >= 4)
            def _():
                wait_store(s - 4, j)

            transpose_add(s, j)
        return 0

    lax.fori_loop(0, COLS // 4, main, 0)

    for s in range(COLS - 2, COLS):
        start_store(s, s % 4)
    for s in range(COLS - 4, COLS):
        wait_store(s, s % 4)


@jax.jit
def _embed(xt, tab):
    k = pl.kernel(
        _body,
        out_type=jax.ShapeDtypeStruct((COLS, D // 8, ROWS // TPW, 8, TPW), jnp.float32),
        mesh=_mesh,
        compiler_params=pltpu.CompilerParams(
            use_tc_tiling_on_sc=False, needs_layout_passes=False),
        scratch_types=[
            pltpu.VMEM((COLS, TPW), jnp.int32),
            pltpu.VMEM((4, TPW, 2 * D), jnp.float32),
            pltpu.VMEM((4, D // 8, 8, TPW), jnp.float32),
            pltpu.SemaphoreType.DMA((4,)),
            pltpu.SemaphoreType.DMA((4,)),
        ],
    )
    return k(xt, tab)


def kernel(x, table):
    xt = x.T
    tab = jnp.pad(table, ((0, 0), (0, D)))
    out6 = _embed(xt, tab)
    return out6.transpose(2, 4, 0, 1, 3).reshape(ROWS, COLS, D)


# speedup vs baseline: 1.4958x; 1.2398x over previous
import math

import jax
import jax.numpy as jnp
from jax import lax
from jax.experimental import pallas as pl
from jax.experimental.pallas import tpu as pltpu
from jax.experimental.pallas import tpu_sc as plsc

VOCAB = 1000000
D = 64
ROWS = 4096
COLS = 200
NC = 2
NS = 16
NW = NC * NS
TPW = ROWS // NW
L = 16
SCALE = math.sqrt(D)

_mesh = plsc.VectorSubcoreMesh(
    core_axis_name="c", subcore_axis_name="s", num_cores=NC, num_subcores=NS
)


def _body(x_hbm, tab_hbm, out_hbm, xq_v, in_v, out_v, gsem, ssem):
    wid = lax.axis_index("s") * NC + lax.axis_index("c")
    a0 = wid * TPW
    pltpu.sync_copy(x_hbm.at[:, pl.ds(a0, TPW)], xq_v)

    def start_gather(s, b):
        pltpu.async_copy(
            tab_hbm.at[xq_v.at[s]], in_v.at[b], gsem.at[b])

    def wait_gather(s, b):
        pltpu.make_async_copy(
            tab_hbm.at[xq_v.at[s]], in_v.at[b], gsem.at[b]).wait()

    def start_store(s, b):
        pltpu.async_copy(out_v.at[b], out_hbm.at[s, pl.ds(0, D // 8), wid], ssem.at[b])

    def wait_store(s, b):
        pltpu.make_async_copy(
            out_v.at[b], out_hbm.at[s, pl.ds(0, D // 8), wid], ssem.at[b]).wait()

    def transpose_add(s, b):
        lanes = lax.broadcasted_iota(jnp.int32, (L,), 0)

        @plsc.parallel_loop(0, (TPW // L) * (D // L), unroll=2)
        def _chunk(i):
            p = i >> 2
            c16 = (i & 3) * L
            rows = lanes + p * L
            for k in range(L):
                cc = ((lanes + k) & (L - 1)) + c16
                vals = plsc.load_gather(in_v.at[b], [rows, cc])
                plsc.store_scatter(
                    out_v.at[b], [cc >> 3, cc & 7, rows], vals + SCALE)


    for b in range(2):
        start_gather(b, b)

    def main(i, _):
        s0 = i * 4
        for j in range(4):
            s = s0 + j

            @pl.when(s >= 2)
            def _():
                start_store(s - 2, (j + 2) % 4)

            @pl.when(s + 2 < COLS)
            def _():
                start_gather(s + 2, (j + 2) % 4)

            wait_gather(s, j)

            @pl.when(s >= 4)
            def _():
                wait_store(s - 4, j)

            transpose_add(s, j)
        return 0

    lax.fori_loop(0, COLS // 4, main, 0)

    for s in range(COLS - 2, COLS):
        start_store(s, s % 4)
    for s in range(COLS - 4, COLS):
        wait_store(s, s % 4)


@jax.jit
def _embed(xt, tab):
    k = pl.kernel(
        _body,
        out_type=jax.ShapeDtypeStruct((COLS, D // 8, ROWS // TPW, 8, TPW), jnp.float32),
        mesh=_mesh,
        compiler_params=pltpu.CompilerParams(
            use_tc_tiling_on_sc=False, needs_layout_passes=False),
        scratch_types=[
            pltpu.VMEM((COLS, TPW), jnp.int32),
            pltpu.VMEM((4, TPW, 2 * D), jnp.float32),
            pltpu.VMEM((4, D // 8, 8, TPW), jnp.float32),
            pltpu.SemaphoreType.DMA((4,)),
            pltpu.SemaphoreType.DMA((4,)),
        ],
    )
    return k(xt, tab)


def kernel(x, table):
    xt = x.T
    tab = jnp.pad(table, ((0, 0), (0, D)))
    out6 = _embed(xt, tab)
    return out6.transpose(2, 4, 0, 1, 3).reshape(ROWS, COLS, D)
